# baseline (speedup 1.0000x reference)
import functools

import jax
import jax.numpy as jnp
from jax import lax
from jax.experimental import pallas as pl
from jax.experimental.pallas import tpu as pltpu
from jax.experimental.pallas import tpu_sc as plsc

_BUCKETS = 100000
_D = 64
_NC = 2
_NS = 16
_NW = _NC * _NS
_CHUNK = 128
_K = 4
_GK = _K * _CHUNK


@functools.partial(jax.jit, static_argnames=("n_total",))
def _sc_gather(ids, table, n_total):
    b_per_w = n_total // _NW
    n_groups = b_per_w // _GK
    mesh = plsc.VectorSubcoreMesh(core_axis_name="c", subcore_axis_name="s")

    @functools.partial(
        pl.kernel,
        out_type=jax.ShapeDtypeStruct((n_total, _D), jnp.float32),
        mesh=mesh,
        scratch_types=[
            pltpu.VMEM((b_per_w,), jnp.int32),
            pltpu.VMEM((_GK, _D), jnp.float32),
            pltpu.VMEM((_GK, _D), jnp.float32),
            pltpu.SemaphoreType.DMA,
            pltpu.SemaphoreType.DMA,
            pltpu.SemaphoreType.DMA,
            pltpu.SemaphoreType.DMA,
        ],
        compiler_params=pltpu.CompilerParams(use_tc_tiling_on_sc=False),
    )
    def k(ids_hbm, table_hbm, out_hbm, idx_v, rows0, rows1,
          gsem0, gsem1, osem0, osem1):
        rows = (rows0, rows1)
        gsem = (gsem0, gsem1)
        osem = (osem0, osem1)

        wid = lax.axis_index("s") * _NC + lax.axis_index("c")
        base = wid * b_per_w

        pltpu.sync_copy(ids_hbm.at[pl.ds(base, b_per_w)], idx_v)

        @pl.loop(0, b_per_w // 16, step=8)
        def _mod(i):
            for j in range(8):
                sl = pl.ds((i + j) * 16, 16)
                x = idx_v[sl]
                for c in (8 * _BUCKETS, 4 * _BUCKETS, 2 * _BUCKETS, _BUCKETS):
                    x = jnp.where(x >= c, x - c, x)
                idx_v[sl] = x

        def gather_descs(g, b):
            return [
                pltpu.make_async_copy(
                    table_hbm.at[idx_v.at[pl.ds(g * _GK + j * _CHUNK, _CHUNK)]],
                    rows[b].at[pl.ds(j * _CHUNK, _CHUNK)],
                    gsem[b],
                )
                for j in range(_K)
            ]

        def out_desc(g, b):
            return pltpu.make_async_copy(
                rows[b], out_hbm.at[pl.ds(base + g * _GK, _GK)], osem[b])

        def fire(g, b):
            for d in gather_descs(g, b):
                d.start()

        fire(0, 0)

        @pl.loop(0, n_groups, step=2)
        def _main(g0):
            for b in range(2):
                g = g0 + b

                @pl.when(g + 1 < n_groups)
                def _fire_next():
                    @pl.when(g >= 1)
                    def _wait_prev_out():
                        out_desc(g - 1, 1 - b).wait()
                    fire(g + 1, 1 - b)

                for d in gather_descs(g, b):
                    d.wait()
                out_desc(g, b).start()

        out_desc(n_groups - 2, 0).wait()
        out_desc(n_groups - 1, 1).wait()

    return k(ids, table)


def _tc_transpose(h2, ns, nb):
    blk = 512

    def body(in_ref, out_ref):
        x = in_ref[0]
        t = x.T
        out_ref[0, :, pl.ds(0, blk // 2)] = t[0:_D]
        out_ref[0, :, pl.ds(blk // 2, blk // 2)] = t[_D:2 * _D]

    return pl.pallas_call(
        body,
        grid=(ns, nb // blk),
        in_specs=[pl.BlockSpec((1, blk // 2, 2 * _D), lambda s, b: (s, b, 0))],
        out_specs=pl.BlockSpec((1, _D, blk), lambda s, b: (s, 0, b)),
        out_shape=jax.ShapeDtypeStruct((ns, _D, nb), jnp.float32),
    )(h2)


def kernel(token_ids, bucket_embeddings):
    nb, ns = token_ids.shape
    n_total = nb * ns
    ids_perm = (
        token_ids.T.astype(jnp.int32)
        .reshape(ns, nb // 512, 2, 256)
        .transpose(0, 1, 3, 2)
        .reshape(n_total)
    )
    out64 = _sc_gather(ids_perm, bucket_embeddings, n_total)
    h2 = out64.reshape(ns, nb // 2, 2 * _D)
    out_t = _tc_transpose(h2, ns, nb)
    return jnp.transpose(out_t, (2, 0, 1))


# speedup vs baseline: 1.9382x; 1.9382x over previous
import functools

import jax
import jax.numpy as jnp
from jax import lax
from jax.experimental import pallas as pl
from jax.experimental.pallas import tpu as pltpu
from jax.experimental.pallas import tpu_sc as plsc

_BUCKETS = 100000
_D = 64
_NC = 2
_NS = 16
_NW = _NC * _NS
_CHUNK = 128
_K = 4
_GK = _K * _CHUNK


@functools.partial(jax.jit, static_argnames=("n_total",))
def _sc_gather(ids, table, n_total):
    b_per_w = n_total // _NW
    n_groups = b_per_w // _GK
    mesh = plsc.VectorSubcoreMesh(core_axis_name="c", subcore_axis_name="s")

    @functools.partial(
        pl.kernel,
        out_type=jax.ShapeDtypeStruct((n_total, _D), jnp.float32),
        mesh=mesh,
        scratch_types=[
            pltpu.VMEM((b_per_w,), jnp.int32),
            pltpu.VMEM((_GK, _D), jnp.float32),
            pltpu.VMEM((_GK, _D), jnp.float32),
            pltpu.SemaphoreType.DMA,
            pltpu.SemaphoreType.DMA,
            pltpu.SemaphoreType.DMA,
            pltpu.SemaphoreType.DMA,
        ],
        compiler_params=pltpu.CompilerParams(use_tc_tiling_on_sc=False),
    )
    def k(ids_hbm, table_hbm, out_hbm, idx_v, rows0, rows1,
          gsem0, gsem1, osem0, osem1):
        rows = (rows0, rows1)
        gsem = (gsem0, gsem1)
        osem = (osem0, osem1)

        wid = lax.axis_index("s") * _NC + lax.axis_index("c")
        base = wid * b_per_w

        pltpu.sync_copy(ids_hbm.at[pl.ds(base, b_per_w)], idx_v)

        @pl.loop(0, b_per_w // 16, step=8)
        def _mod(i):
            for j in range(8):
                sl = pl.ds((i + j) * 16, 16)
                x = idx_v[sl]
                for c in (8 * _BUCKETS, 4 * _BUCKETS, 2 * _BUCKETS, _BUCKETS):
                    x = jnp.where(x >= c, x - c, x)
                idx_v[sl] = x

        def gather_descs(g, b):
            return [
                pltpu.make_async_copy(
                    table_hbm.at[idx_v.at[pl.ds(g * _GK + j * _CHUNK, _CHUNK)]],
                    rows[b].at[pl.ds(j * _CHUNK, _CHUNK)],
                    gsem[b],
                )
                for j in range(_K)
            ]

        def out_desc(g, b):
            return pltpu.make_async_copy(
                rows[b], out_hbm.at[pl.ds(base + g * _GK, _GK)], osem[b])

        def fire(g, b):
            for d in gather_descs(g, b):
                d.start()

        fire(0, 0)

        @pl.loop(0, n_groups, step=2)
        def _main(g0):
            for b in range(2):
                g = g0 + b

                @pl.when(g + 1 < n_groups)
                def _fire_next():
                    @pl.when(g >= 1)
                    def _wait_prev_out():
                        out_desc(g - 1, 1 - b).wait()
                    fire(g + 1, 1 - b)

                for d in gather_descs(g, b):
                    d.wait()
                out_desc(g, b).start()

        out_desc(n_groups - 2, 0).wait()
        out_desc(n_groups - 1, 1).wait()

    return k(ids, table)


def kernel(token_ids, bucket_embeddings):
    b, s = token_ids.shape
    n_total = b * s
    ids = token_ids.reshape(n_total).astype(jnp.int32)
    out = _sc_gather(ids, bucket_embeddings, n_total)
    return out.reshape(b, s, _D)
